# baseline (speedup 1.0000x reference)
import functools

import jax
import jax.numpy as jnp
import numpy as np
from jax import lax
from jax.experimental import pallas as pl
from jax.experimental.pallas import tpu as pltpu
from jax.experimental.pallas import tpu_sc as plsc

_SILU_NORM = 1.679177
_NUM_NEIGHBORS = 32.0

_NC = 2
_NS = 16
_LANES = 16


def _x_body(ni_ref, na_ref, w_ref, out_ref):
    out_ref[...] = jnp.dot(ni_ref[...], w_ref[...]) * na_ref[...]


def _mlp_body(est_ref, ea_ref, w0_ref, w1_ref, w2_ref, out_ref):
    dn = (((0,), (0,)), ((), ()))
    h = lax.dot_general(
        est_ref[...].astype(jnp.bfloat16), w0_ref[...], dn,
        preferred_element_type=jnp.float32,
    )
    h = (jax.nn.silu(h) * _SILU_NORM).astype(jnp.bfloat16)
    h = jnp.dot(h, w1_ref[...], preferred_element_type=jnp.float32)
    h = (jax.nn.silu(h) * _SILU_NORM).astype(jnp.bfloat16)
    h = jnp.dot(h, w2_ref[...], preferred_element_type=jnp.float32)
    ea_col = jnp.swapaxes(ea_ref[...], 0, 1)
    out_ref[...] = h * ea_col


def _out_body(pa_ref, pb_ref, na_ref, w_ref, out_ref, *, n):
    s = (pa_ref[:n, :] + pa_ref[n:, :]) + (pb_ref[:n, :] + pb_ref[n:, :])
    out_ref[...] = jnp.dot(s, w_ref[...]) * na_ref[...]


def _sc_body(
    w_hbm, src_hbm, dst_hbm, x_hbm, out_hbm,
    srcv, dstq, wv0, wv1, gv0, gv1, acc,
    sem_g0, sem_g1, sem_w0, sem_w1, sem_s0, sem_s1, sem_i0, sem_i1,
    *, n, d, epw, chunk, nchunks, seg, zero_acc,
):
    c = lax.axis_index("c")
    s = lax.axis_index("s")
    wid = c * _NS + s
    wbase = wid * epw
    nseg = nchunks // seg
    wvs = [wv0, wv1]
    gvs = [gv0, gv1]
    sem_g = [sem_g0, sem_g1]
    sem_w = [sem_w0, sem_w1]
    sem_s = [sem_s0, sem_s1]
    sem_i = [sem_i0, sem_i1]

    rps = (n // _NS) // 8 * 8
    rlast = n - (_NS - 1) * rps
    base = pl.multiple_of(s * rps, 8)
    zero16 = jnp.zeros((_LANES,), jnp.float32)

    if zero_acc:
        def _zrow(r, _):
            for j in range(d // _LANES):
                wv0[r, pl.ds(j * _LANES, _LANES)] = zero16
            return 0

        lax.fori_loop(0, chunk, _zrow, 0)

        def _zero_rows(nrows):
            for k in range(nrows // chunk):
                pltpu.sync_copy(
                    wv0,
                    acc.at[
                        pl.ds(pl.multiple_of(base + k * chunk, 8), chunk), :
                    ],
                )
            rem = nrows % chunk
            if rem:
                pltpu.sync_copy(
                    wv0.at[pl.ds(0, rem), :],
                    acc.at[
                        pl.ds(pl.multiple_of(base + nrows - rem, 8), rem), :
                    ],
                )

        @pl.when(s < _NS - 1)
        def _():
            _zero_rows(rps)

        @pl.when(s == _NS - 1)
        def _():
            _zero_rows(rlast)

        plsc.subcore_barrier()

    def _woff(g):
        return pl.multiple_of(wbase + g * chunk, 8)

    def _gslice(l):
        return srcv.at[pl.ds(pl.multiple_of(l * chunk, 8), chunk)]

    def _issue_g(l, b):
        pltpu.async_copy(x_hbm.at[_gslice(l)], gvs[b], sem_g[b])

    def _issue_w(g, b):
        pltpu.async_copy(w_hbm.at[pl.ds(_woff(g), chunk), :], wvs[b], sem_w[b])

    def _issue_gw(l, g, b):
        _issue_g(l, b)
        _issue_w(g, b)

    def _wait_gw(l, g, b):
        pltpu.make_async_copy(x_hbm.at[_gslice(l)], gvs[b], sem_g[b]).wait()
        pltpu.make_async_copy(
            w_hbm.at[pl.ds(_woff(g), chunk), :], wvs[b], sem_w[b]
        ).wait()

    def _issue_idx(g, b):
        pltpu.async_copy(
            dst_hbm.at[pl.ds(_woff(g), chunk)],
            dstq.at[lax.rem(g, 4)],
            sem_i[b],
        )

    def _wait_idx(g, b):
        pltpu.make_async_copy(
            dst_hbm.at[pl.ds(_woff(g), chunk)],
            dstq.at[lax.rem(g, 4)],
            sem_i[b],
        ).wait()

    def _issue_scat(g, b):
        pltpu.async_copy(
            wvs[b], acc.at[dstq.at[lax.rem(g, 4)]], sem_s[b], add=True
        )

    def _wait_scat(g, b):
        pltpu.make_async_copy(
            wvs[b], acc.at[dstq.at[lax.rem(g, 4)]], sem_s[b]
        ).wait()

    def _compute(b):
        wv, gv = wvs[b], gvs[b]

        def _prow(t, _):
            for i in (0, 1):
                r = 2 * t + i
                for j in range(d // _LANES):
                    sl = pl.ds(j * _LANES, _LANES)
                    wv[r, sl] = wv[r, sl] * gv[r, sl]
            return 0

        lax.fori_loop(0, chunk // 2, _prow, 0)

    def _chunk_body(l, g, b, first, issue_next, issue_idx2, wait_idx):
        _wait_gw(l, g, b)
        if issue_next:
            _issue_g(l + 1, 1 - b)
        if not first:
            _wait_scat(g - 1, 1 - b)
        if issue_next:
            _issue_w(g + 1, 1 - b)
        if wait_idx:
            _wait_idx(g, b)
        _compute(b)
        _issue_scat(g, b)
        if issue_idx2:
            _issue_idx(g + 2, b)

    for m in range(nseg):
        g0 = m * seg
        pltpu.sync_copy(
            src_hbm.at[pl.ds(_woff(g0), seg * chunk)],
            srcv.at[pl.ds(0, seg * chunk)],
        )
        for gp in (g0, g0 + 1):
            pltpu.sync_copy(
                dst_hbm.at[pl.ds(_woff(gp), chunk)], dstq.at[lax.rem(gp, 4)]
            )
        _issue_gw(0, g0, 0)
        _chunk_body(0, g0, 0, True, True, True, False)
        _chunk_body(1, g0 + 1, 1, False, True, True, False)

        def _pair(t, _):
            l = 2 + 2 * t
            g = g0 + l
            _chunk_body(l, g, 0, False, True, True, True)
            _chunk_body(l + 1, g + 1, 1, False, True, True, True)
            return 0

        lax.fori_loop(0, (seg - 4) // 2, _pair, 0)
        _chunk_body(seg - 2, g0 + seg - 2, 0, False, True, False, True)
        _chunk_body(seg - 1, g0 + seg - 1, 1, False, False, False, True)
        _wait_scat(g0 + seg - 1, 1)

    if nchunks > nseg * seg:
        gl = nseg * seg
        pltpu.sync_copy(
            src_hbm.at[pl.ds(_woff(gl), chunk)], srcv.at[pl.ds(0, chunk)]
        )
        pltpu.sync_copy(
            dst_hbm.at[pl.ds(_woff(gl), chunk)], dstq.at[lax.rem(gl, 4)]
        )
        _issue_gw(0, gl, 0)
        _wait_gw(0, gl, 0)
        _compute(0)
        pltpu.sync_copy(wv0, acc.at[dstq.at[lax.rem(gl, 4)]], add=True)

    plsc.subcore_barrier()
    obase = pl.multiple_of(c * n + base, 8)

    @pl.when(s < _NS - 1)
    def _():
        pltpu.sync_copy(
            acc.at[pl.ds(base, rps), :], out_hbm.at[pl.ds(obase, rps), :]
        )

    @pl.when(s == _NS - 1)
    def _():
        pltpu.sync_copy(
            acc.at[pl.ds(base, rlast), :], out_hbm.at[pl.ds(obase, rlast), :]
        )


def kernel(node_input, node_attr_input, node_attr_output, edge_src, edge_dst,
           edge_attr, edge_scalar_attr, W_lin1, fc_w0, fc_w1, fc_w2, W_lin2):
    n, d = node_input.shape
    e = edge_src.shape[0]
    nes = edge_scalar_attr.shape[1]
    radial = fc_w0.shape[1]

    w1s = W_lin1 / np.sqrt(d)
    w0s = (fc_w0 / np.sqrt(nes)).astype(jnp.bfloat16)
    w1m = (fc_w1 / np.sqrt(radial)).astype(jnp.bfloat16)
    w2m = (fc_w2 / np.sqrt(radial)).astype(jnp.bfloat16)
    w2s = W_lin2 / (np.sqrt(d) * np.sqrt(_NUM_NEIGHBORS))

    edge_src = edge_src.astype(jnp.int32)
    edge_dst = edge_dst.astype(jnp.int32)

    x = pl.pallas_call(
        _x_body,
        out_shape=jax.ShapeDtypeStruct((n, d), jnp.float32),
    )(node_input, node_attr_input, w1s)

    est = edge_scalar_attr.T
    ea2d = edge_attr.reshape(1, e)

    nw = _NC * _NS
    chunk = 80
    seg = 62
    unit = nw * chunk
    e_a = seg * unit
    blk = 2560
    mesh = plsc.VectorSubcoreMesh(
        core_axis_name="c", subcore_axis_name="s",
        num_cores=_NC, num_subcores=_NS,
    )

    def _mlp_part(e_part, blk, blk_off):
        grid = e_part // blk
        return pl.pallas_call(
            _mlp_body,
            grid=(grid,),
            in_specs=[
                pl.BlockSpec((nes, blk), lambda i: (0, i + blk_off)),
                pl.BlockSpec((1, blk), lambda i: (0, i + blk_off)),
                pl.BlockSpec((nes, radial), lambda i: (0, 0)),
                pl.BlockSpec((radial, radial), lambda i: (0, 0)),
                pl.BlockSpec((radial, d), lambda i: (0, 0)),
            ],
            out_specs=pl.BlockSpec((blk, d), lambda i: (i, 0)),
            out_shape=jax.ShapeDtypeStruct((e_part, d), jnp.float32),
        )(est, ea2d, w0s, w1m, w2m)

    w_a = _mlp_part(e_a, 19840, 0)
    w_b = _mlp_part(e - e_a, blk, e_a // blk)

    def _sc_part(w_h, src_h, dst_h, epw):
        nchunks = epw // chunk
        sc_fn = functools.partial(
            _sc_body, n=n, d=d, epw=epw, chunk=chunk, nchunks=nchunks,
            seg=seg, zero_acc=True,
        )
        return pl.kernel(
            sc_fn,
            out_type=jax.ShapeDtypeStruct((_NC * n, d), jnp.float32),
            mesh=mesh,
            scratch_types=[
                pltpu.VMEM((seg * chunk,), jnp.int32),
                pltpu.VMEM((4, chunk), jnp.int32),
                pltpu.VMEM((chunk, d), jnp.float32),
                pltpu.VMEM((chunk, d), jnp.float32),
                pltpu.VMEM((chunk, d), jnp.float32),
                pltpu.VMEM((chunk, d), jnp.float32),
                pltpu.VMEM_SHARED((n, d), jnp.float32),
                pltpu.SemaphoreType.DMA,
                pltpu.SemaphoreType.DMA,
                pltpu.SemaphoreType.DMA,
                pltpu.SemaphoreType.DMA,
                pltpu.SemaphoreType.DMA,
                pltpu.SemaphoreType.DMA,
                pltpu.SemaphoreType.DMA,
                pltpu.SemaphoreType.DMA,
            ],
        )(w_h, src_h, dst_h, x)

    parts_a = _sc_part(w_a, edge_src[:e_a], edge_dst[:e_a], e_a // nw)
    parts_b = _sc_part(
        w_b, edge_src[e_a:], edge_dst[e_a:], (e - e_a) // nw
    )

    out = pl.pallas_call(
        functools.partial(_out_body, n=n),
        out_shape=jax.ShapeDtypeStruct((n, d), jnp.float32),
    )(parts_a, parts_b, node_attr_output, w2s)
    return out


# speedup vs baseline: 1.0168x; 1.0168x over previous
import functools

import jax
import jax.numpy as jnp
import numpy as np
from jax import lax
from jax.experimental import pallas as pl
from jax.experimental.pallas import tpu as pltpu
from jax.experimental.pallas import tpu_sc as plsc

_SILU_NORM = 1.679177
_NUM_NEIGHBORS = 32.0

_NC = 2
_NS = 16
_LANES = 16


def _x_body(ni_ref, na_ref, w_ref, out_ref):
    out_ref[...] = jnp.dot(ni_ref[...], w_ref[...]) * na_ref[...]


def _mlp_body(est_ref, ea_ref, w0_ref, w1_ref, w2_ref, out_ref):
    dn = (((0,), (0,)), ((), ()))
    h = lax.dot_general(
        est_ref[...].astype(jnp.bfloat16), w0_ref[...], dn,
        preferred_element_type=jnp.float32,
    )
    h = (jax.nn.silu(h) * _SILU_NORM).astype(jnp.bfloat16)
    h = jnp.dot(h, w1_ref[...], preferred_element_type=jnp.float32)
    h = (jax.nn.silu(h) * _SILU_NORM).astype(jnp.bfloat16)
    h = jnp.dot(h, w2_ref[...], preferred_element_type=jnp.float32)
    ea_col = jnp.swapaxes(ea_ref[...], 0, 1)
    out_ref[...] = h * ea_col


def _out_body(pa_ref, pb_ref, na_ref, w_ref, out_ref, *, n):
    s = (pa_ref[:n, :] + pa_ref[n:, :]) + (pb_ref[:n, :] + pb_ref[n:, :])
    out_ref[...] = jnp.dot(s, w_ref[...]) * na_ref[...]


def _sc_body(
    w_hbm, src_hbm, dst_hbm, x_hbm, out_hbm,
    srcv, dstq, wv0, wv1, gv0, gv1, acc,
    sem_g0, sem_g1, sem_w0, sem_w1, sem_s0, sem_s1, sem_i0, sem_i1,
    *, n, d, epw, chunk, nchunks, seg, zero_acc,
):
    c = lax.axis_index("c")
    s = lax.axis_index("s")
    wid = c * _NS + s
    wbase = wid * epw
    nseg = nchunks // seg
    wvs = [wv0, wv1]
    gvs = [gv0, gv1]
    sem_g = [sem_g0, sem_g1]
    sem_w = [sem_w0, sem_w1]
    sem_s = [sem_s0, sem_s1]
    sem_i = [sem_i0, sem_i1]

    rps = (n // _NS) // 8 * 8
    rlast = n - (_NS - 1) * rps
    base = pl.multiple_of(s * rps, 8)
    zero16 = jnp.zeros((_LANES,), jnp.float32)

    if zero_acc:
        def _zrow(r, _):
            for j in range(d // _LANES):
                wv0[r, pl.ds(j * _LANES, _LANES)] = zero16
            return 0

        lax.fori_loop(0, chunk, _zrow, 0)

        def _zero_rows(nrows):
            for k in range(nrows // chunk):
                pltpu.sync_copy(
                    wv0,
                    acc.at[
                        pl.ds(pl.multiple_of(base + k * chunk, 8), chunk), :
                    ],
                )
            rem = nrows % chunk
            if rem:
                pltpu.sync_copy(
                    wv0.at[pl.ds(0, rem), :],
                    acc.at[
                        pl.ds(pl.multiple_of(base + nrows - rem, 8), rem), :
                    ],
                )

        @pl.when(s < _NS - 1)
        def _():
            _zero_rows(rps)

        @pl.when(s == _NS - 1)
        def _():
            _zero_rows(rlast)

        plsc.subcore_barrier()

    def _woff(g):
        return pl.multiple_of(wbase + g * chunk, 8)

    def _gslice(l):
        return srcv.at[pl.ds(pl.multiple_of(l * chunk, 8), chunk)]

    def _issue_g(l, b):
        pltpu.async_copy(x_hbm.at[_gslice(l)], gvs[b], sem_g[b])

    def _issue_w(g, b):
        pltpu.async_copy(w_hbm.at[pl.ds(_woff(g), chunk), :], wvs[b], sem_w[b])

    def _issue_gw(l, g, b):
        _issue_g(l, b)
        _issue_w(g, b)

    def _wait_gw(l, g, b):
        pltpu.make_async_copy(x_hbm.at[_gslice(l)], gvs[b], sem_g[b]).wait()
        pltpu.make_async_copy(
            w_hbm.at[pl.ds(_woff(g), chunk), :], wvs[b], sem_w[b]
        ).wait()

    def _issue_idx(g, b):
        pltpu.async_copy(
            dst_hbm.at[pl.ds(_woff(g), chunk)],
            dstq.at[lax.rem(g, 4)],
            sem_i[b],
        )

    def _wait_idx(g, b):
        pltpu.make_async_copy(
            dst_hbm.at[pl.ds(_woff(g), chunk)],
            dstq.at[lax.rem(g, 4)],
            sem_i[b],
        ).wait()

    def _issue_scat(g, b):
        pltpu.async_copy(
            wvs[b], acc.at[dstq.at[lax.rem(g, 4)]], sem_s[b], add=True
        )

    def _wait_scat(g, b):
        pltpu.make_async_copy(
            wvs[b], acc.at[dstq.at[lax.rem(g, 4)]], sem_s[b]
        ).wait()

    def _compute(b):
        wv, gv = wvs[b], gvs[b]

        def _prow(t, _):
            for i in (0, 1):
                r = 2 * t + i
                for j in range(d // _LANES):
                    sl = pl.ds(j * _LANES, _LANES)
                    wv[r, sl] = wv[r, sl] * gv[r, sl]
            return 0

        lax.fori_loop(0, chunk // 2, _prow, 0)

    def _chunk_body(l, g, b, first, issue_next, issue_idx2, wait_idx):
        _wait_gw(l, g, b)
        if issue_next:
            _issue_g(l + 1, 1 - b)
        if not first:
            _wait_scat(g - 1, 1 - b)
        if issue_next:
            _issue_w(g + 1, 1 - b)
        if wait_idx:
            _wait_idx(g, b)
        _compute(b)
        _issue_scat(g, b)
        if issue_idx2:
            _issue_idx(g + 2, b)

    for m in range(nseg):
        g0 = m * seg
        pltpu.sync_copy(
            src_hbm.at[pl.ds(_woff(g0), seg * chunk)],
            srcv.at[pl.ds(0, seg * chunk)],
        )
        for gp in (g0, g0 + 1):
            pltpu.sync_copy(
                dst_hbm.at[pl.ds(_woff(gp), chunk)], dstq.at[lax.rem(gp, 4)]
            )
        _issue_gw(0, g0, 0)
        _chunk_body(0, g0, 0, True, True, True, False)
        _chunk_body(1, g0 + 1, 1, False, True, True, False)

        def _pair(t, _):
            l = 2 + 2 * t
            g = g0 + l
            _chunk_body(l, g, 0, False, True, True, True)
            _chunk_body(l + 1, g + 1, 1, False, True, True, True)
            return 0

        lax.fori_loop(0, (seg - 4) // 2, _pair, 0)
        _chunk_body(seg - 2, g0 + seg - 2, 0, False, True, False, True)
        _chunk_body(seg - 1, g0 + seg - 1, 1, False, False, False, True)
        _wait_scat(g0 + seg - 1, 1)

    if nchunks > nseg * seg:
        gl = nseg * seg
        pltpu.sync_copy(
            src_hbm.at[pl.ds(_woff(gl), chunk)], srcv.at[pl.ds(0, chunk)]
        )
        pltpu.sync_copy(
            dst_hbm.at[pl.ds(_woff(gl), chunk)], dstq.at[lax.rem(gl, 4)]
        )
        _issue_gw(0, gl, 0)
        _wait_gw(0, gl, 0)
        _compute(0)
        pltpu.sync_copy(wv0, acc.at[dstq.at[lax.rem(gl, 4)]], add=True)

    plsc.subcore_barrier()
    obase = pl.multiple_of(c * n + base, 8)

    @pl.when(s < _NS - 1)
    def _():
        pltpu.sync_copy(
            acc.at[pl.ds(base, rps), :], out_hbm.at[pl.ds(obase, rps), :]
        )

    @pl.when(s == _NS - 1)
    def _():
        pltpu.sync_copy(
            acc.at[pl.ds(base, rlast), :], out_hbm.at[pl.ds(obase, rlast), :]
        )


def kernel(node_input, node_attr_input, node_attr_output, edge_src, edge_dst,
           edge_attr, edge_scalar_attr, W_lin1, fc_w0, fc_w1, fc_w2, W_lin2):
    n, d = node_input.shape
    e = edge_src.shape[0]
    nes = edge_scalar_attr.shape[1]
    radial = fc_w0.shape[1]

    w1s = W_lin1 / np.sqrt(d)
    w0s = (fc_w0 / np.sqrt(nes)).astype(jnp.bfloat16)
    w1m = (fc_w1 / np.sqrt(radial)).astype(jnp.bfloat16)
    w2m = (fc_w2 / np.sqrt(radial)).astype(jnp.bfloat16)
    w2s = W_lin2 / (np.sqrt(d) * np.sqrt(_NUM_NEIGHBORS))

    edge_src = edge_src.astype(jnp.int32)
    edge_dst = edge_dst.astype(jnp.int32)

    x = pl.pallas_call(
        _x_body,
        out_shape=jax.ShapeDtypeStruct((n, d), jnp.float32),
    )(node_input, node_attr_input, w1s)

    est = edge_scalar_attr.T
    ea2d = edge_attr.reshape(1, e)

    nw = _NC * _NS
    chunk = 80
    seg = 62
    unit = nw * chunk
    e_a = seg * unit
    blk = 2560
    mesh = plsc.VectorSubcoreMesh(
        core_axis_name="c", subcore_axis_name="s",
        num_cores=_NC, num_subcores=_NS,
    )

    def _mlp_part(e_part, blk, blk_off):
        grid = e_part // blk
        return pl.pallas_call(
            _mlp_body,
            grid=(grid,),
            in_specs=[
                pl.BlockSpec((nes, blk), lambda i: (0, i + blk_off)),
                pl.BlockSpec((1, blk), lambda i: (0, i + blk_off)),
                pl.BlockSpec((nes, radial), lambda i: (0, 0)),
                pl.BlockSpec((radial, radial), lambda i: (0, 0)),
                pl.BlockSpec((radial, d), lambda i: (0, 0)),
            ],
            out_specs=pl.BlockSpec((blk, d), lambda i: (i, 0)),
            out_shape=jax.ShapeDtypeStruct((e_part, d), jnp.float32),
        )(est, ea2d, w0s, w1m, w2m)

    w_a = _mlp_part(e_a, blk, 0)
    w_b = _mlp_part(e - e_a, blk, e_a // blk)

    def _sc_part(w_h, src_h, dst_h, epw):
        nchunks = epw // chunk
        sc_fn = functools.partial(
            _sc_body, n=n, d=d, epw=epw, chunk=chunk, nchunks=nchunks,
            seg=seg, zero_acc=True,
        )
        return pl.kernel(
            sc_fn,
            out_type=jax.ShapeDtypeStruct((_NC * n, d), jnp.float32),
            mesh=mesh,
            scratch_types=[
                pltpu.VMEM((seg * chunk,), jnp.int32),
                pltpu.VMEM((4, chunk), jnp.int32),
                pltpu.VMEM((chunk, d), jnp.float32),
                pltpu.VMEM((chunk, d), jnp.float32),
                pltpu.VMEM((chunk, d), jnp.float32),
                pltpu.VMEM((chunk, d), jnp.float32),
                pltpu.VMEM_SHARED((n, d), jnp.float32),
                pltpu.SemaphoreType.DMA,
                pltpu.SemaphoreType.DMA,
                pltpu.SemaphoreType.DMA,
                pltpu.SemaphoreType.DMA,
                pltpu.SemaphoreType.DMA,
                pltpu.SemaphoreType.DMA,
                pltpu.SemaphoreType.DMA,
                pltpu.SemaphoreType.DMA,
            ],
        )(w_h, src_h, dst_h, x)

    parts_a = _sc_part(w_a, edge_src[:e_a], edge_dst[:e_a], e_a // nw)
    parts_b = _sc_part(
        w_b, edge_src[e_a:], edge_dst[e_a:], (e - e_a) // nw
    )

    out = pl.pallas_call(
        functools.partial(_out_body, n=n),
        out_shape=jax.ShapeDtypeStruct((n, d), jnp.float32),
    )(parts_a, parts_b, node_attr_output, w2s)
    return out
